# baseline (speedup 1.0000x reference)
import functools

import numpy as np

import jax
import jax.numpy as jnp
from jax.experimental import pallas as pl
from jax.experimental.pallas import tpu as pltpu

SB = 12
CPAD = 16

def _class_taps():
    table = []
    for r in range(8):
        hp, dh = r // 4, r % 4
        entries = []
        for ph in range(2):
            kappa = (hp + ph) % 2
            delta = (ph + kappa) // 2
            kh = 4 * kappa + dh
            for pw, dj in ((0, 0), (1, 0), (1, 1)):
                entries.append((ph * 2 + pw, delta, dj, kh, pw))
        table.append(entries)
    return table

_TABLE = _class_taps()


def _layout_kernel(x_ref, o_ref, *, nb, C):
    f32 = jnp.float32
    wps = []
    for wp in range(12):
        lo, hi = 8 * wp, 8 * wp + 8
        cs = []
        for c in range(C):
            v = x_ref[0, :, c, :, :]
            if hi <= 90:
                cs.append(v[:, :, lo:hi])
            else:
                cs.append(jnp.concatenate(
                    [v[:, :, lo:90], jnp.zeros((nb, 84, hi - 90), f32)],
                    axis=-1))
        wps.append(jnp.concatenate(cs, axis=-1))
    xw = jnp.concatenate(wps, axis=-1)
    xw = jnp.concatenate([xw, jnp.zeros((nb, 12, 384), f32)], axis=1)
    o_ref[0] = xw.reshape(nb, 12, 8, 3, 128)


def _fused_kernel(x_ref, p_ref, w1c_ref, b1_ref, w2_ref, b2_ref, w3_ref,
                  b3_ref, wh_ref, bh_ref, wq_ref, bq_ref, o_ref,
                  a0, a1, a2, a3, *, nb):
    nbr = nb * SB
    n1 = 11 * nbr
    f32 = jnp.float32
    accs = (a0, a1, a2, a3)
    k = pl.program_id(1)

    parts = [x_ref[0, :, 0, wp // 4, 32 * (wp % 4):32 * (wp % 4 + 1)]
             for wp in range(12)]
    parts.append(jnp.zeros((CPAD, 32), f32))
    xflat = jnp.concatenate(parts, axis=0)

    @pl.when(k == 0)
    def _():
        for a in accs:
            a[...] = jnp.zeros_like(a)

    for kk in range(8):
        @pl.when(k == kk)
        def _(kk=kk):
            wtap = jnp.dot(p_ref[kk * 192:(kk + 1) * 192, :], w1c_ref[...],
                           preferred_element_type=f32)
            for j, (ocls, delta, dj, _, _) in enumerate(_TABLE[kk]):
                shift = dj * nbr + delta
                accs[ocls][...] += jnp.dot(
                    xflat[shift:shift + n1, :], wtap[j * 32:(j + 1) * 32, :],
                    preferred_element_type=f32)

    @pl.when(k == 7)
    def _():
        b1 = b1_ref[...]
        zpad1 = jnp.zeros((CPAD, 32), f32)
        y1_parts = []
        for a in accs:
            y1_parts.append(jnp.maximum(a[...] + b1, 0.0))
            y1_parts.append(zpad1)
        y1 = jnp.concatenate(y1_parts, axis=0)
        cstride = n1 + CPAD

        n2 = 9 * nbr
        w2 = w2_ref[...]
        acc2 = None
        for kh in range(4):
            for kw in range(4):
                ph, a_ = kh % 2, kh // 2
                pw, b_ = kw % 2, kw // 2
                start = (ph * 2 + pw) * cstride + b_ * nbr + a_
                tap = kh * 4 + kw
                d = jnp.dot(y1[start:start + n2, :],
                            w2[tap * 32:(tap + 1) * 32, :],
                            preferred_element_type=f32)
                acc2 = d if acc2 is None else acc2 + d
        y2 = jnp.maximum(acc2 + b2_ref[...], 0.0)
        y2 = jnp.concatenate([y2, jnp.zeros((CPAD, 64), f32)], axis=0)

        n3 = 7 * nbr
        w3 = w3_ref[...]
        acc3 = None
        for kh in range(3):
            for kw in range(3):
                start = kw * nbr + kh
                tap = kh * 3 + kw
                d = jnp.dot(y2[start:start + n3, :],
                            w3[tap * 64:(tap + 1) * 64, :],
                            preferred_element_type=f32)
                acc3 = d if acc3 is None else acc3 + d
        y3 = jnp.maximum(acc3 + b3_ref[...], 0.0)

        pieces = [y3[t * nbr:(t + 1) * nbr].reshape(nb, SB, 64)
                  for t in range(7)]
        feat = jnp.concatenate(
            [pieces[t][:, s:s + 1, :] for s in range(7) for t in range(7)],
            axis=1).reshape(nb, 49 * 64)
        feat = jnp.concatenate([feat, jnp.zeros((nb, 64), f32)], axis=1)

        h = jnp.maximum(
            jnp.dot(feat.astype(jnp.bfloat16), wh_ref[...],
                    preferred_element_type=f32) + bh_ref[...], 0.0)
        q = jnp.dot(h, wq_ref[...], preferred_element_type=f32) + bq_ref[...]
        o_ref[0] = q


def kernel(x_nchw, conv1_w, conv1_b, conv2_w, conv2_b, conv3_w, conv3_b,
           sel, wh, bh, wq, bq):
    B = x_nchw.shape[0]
    C = x_nchw.shape[1]
    A = wq.shape[1]
    nb = B // 2

    H, W = x_nchw.shape[2], x_nchw.shape[3]
    x = pl.pallas_call(
        functools.partial(_layout_kernel, nb=nb, C=C),
        out_shape=jax.ShapeDtypeStruct((2, nb, 12, 8, 3, 128), jnp.float32),
        grid=(2,),
        in_specs=[pl.BlockSpec((1, nb, C, H, W), lambda i: (i, 0, 0, 0, 0))],
        out_specs=pl.BlockSpec((1, nb, 12, 8, 3, 128),
                               lambda i: (i, 0, 0, 0, 0, 0)),
        compiler_params=pltpu.CompilerParams(
            dimension_semantics=("parallel",)),
    )(x_nchw.reshape(2, nb, C, H, W).astype(jnp.float32))
    x = x.reshape(2, nb * SB, 8, 3, 128)

    psel = np.zeros((48 * 32, 8 * 8 * C), np.float32)
    for r in range(8):
        for j, (_, _, dj, kh, pw) in enumerate(_TABLE[r]):
            t = r * 6 + j
            for w8 in range(8):
                kw = w8 + 8 * dj - 4 * pw
                if 0 <= kw < 8:
                    for c in range(C):
                        psel[t * 32 + c * 8 + w8, (kh * 8 + kw) * C + c] = 1.0

    args = (x, jnp.asarray(psel), conv1_w, conv1_b, conv2_w, conv2_b,
            conv3_w, conv3_b, wh, bh, wq, bq)
    in_specs = [
        pl.BlockSpec((1, nb * SB, 1, 3, 128), lambda i, k: (i, 0, k, 0, 0)),
    ] + [pl.BlockSpec(a.shape, lambda i, k: (0,) * a.ndim) for a in args[1:]]

    n1 = 11 * nb * SB
    out = pl.pallas_call(
        functools.partial(_fused_kernel, nb=nb),
        out_shape=jax.ShapeDtypeStruct((2, nb, A), jnp.float32),
        grid=(2, 8),
        in_specs=in_specs,
        out_specs=pl.BlockSpec((1, nb, A), lambda i, k: (i, 0, 0)),
        scratch_shapes=[pltpu.VMEM((n1, 32), jnp.float32) for _ in range(4)],
        compiler_params=pltpu.CompilerParams(
            dimension_semantics=("parallel", "arbitrary")),
    )(*args)
    return out.reshape(B, A)


# speedup vs baseline: 1.1328x; 1.1328x over previous
import functools

import numpy as np

import jax
import jax.numpy as jnp
from jax.experimental import pallas as pl
from jax.experimental.pallas import tpu as pltpu

SB = 12
CPAD = 16

def _class_taps():
    table = []
    for r in range(8):
        hp, dh = r // 4, r % 4
        entries = []
        for ph in range(2):
            kappa = (hp + ph) % 2
            delta = (ph + kappa) // 2
            kh = 4 * kappa + dh
            for pw, dj in ((0, 0), (1, 0), (1, 1)):
                entries.append((ph * 2 + pw, delta, dj, kh, pw))
        table.append(entries)
    return table

_TABLE = _class_taps()


def _layout_kernel(x_ref, o_ref, *, nb, C):
    f32 = jnp.float32
    wps = []
    for wp in range(12):
        lo, hi = 8 * wp, 8 * wp + 8
        cs = []
        for c in range(C):
            v = x_ref[0, :, c, :, :]
            if hi <= 90:
                cs.append(v[:, :, lo:hi])
            else:
                cs.append(jnp.concatenate(
                    [v[:, :, lo:90], jnp.zeros((nb, 84, hi - 90), f32)],
                    axis=-1))
        wps.append(jnp.concatenate(cs, axis=-1))
    xw = jnp.concatenate(wps, axis=-1)
    xw = jnp.concatenate([xw, jnp.zeros((nb, 12, 384), f32)], axis=1)
    o_ref[0] = xw.reshape(nb, 12, 8, 3, 128)


def _fused_kernel(x_ref, p_ref, w1c_ref, b1_ref, w2_ref, b2_ref, w3_ref,
                  b3_ref, wh_ref, bh_ref, wq_ref, bq_ref, o_ref, *, nb):
    nbr = nb * SB
    n1 = 11 * nbr
    f32 = jnp.float32

    wtaps = jnp.dot(p_ref[...], w1c_ref[...],
                    preferred_element_type=f32)
    accs = [None, None, None, None]
    for kk in range(8):
        parts = [x_ref[0, :, kk, wp // 4, 32 * (wp % 4):32 * (wp % 4 + 1)]
                 for wp in range(12)]
        parts.append(jnp.zeros((CPAD, 32), f32))
        xflat = jnp.concatenate(parts, axis=0)
        for j, (ocls, delta, dj, _, _) in enumerate(_TABLE[kk]):
            shift = dj * nbr + delta
            t = kk * 6 + j
            d = jnp.dot(xflat[shift:shift + n1, :],
                        wtaps[t * 32:(t + 1) * 32, :],
                        preferred_element_type=f32)
            accs[ocls] = d if accs[ocls] is None else accs[ocls] + d

    if True:
        b1 = b1_ref[...]
        zpad1 = jnp.zeros((CPAD, 32), f32)
        y1_parts = []
        for a in accs:
            y1_parts.append(jnp.maximum(a + b1, 0.0))
            y1_parts.append(zpad1)
        y1 = jnp.concatenate(y1_parts, axis=0)
        cstride = n1 + CPAD

        n2 = 9 * nbr
        w2 = w2_ref[...]
        acc2 = None
        for kh in range(4):
            for kw in range(4):
                ph, a_ = kh % 2, kh // 2
                pw, b_ = kw % 2, kw // 2
                start = (ph * 2 + pw) * cstride + b_ * nbr + a_
                tap = kh * 4 + kw
                d = jnp.dot(y1[start:start + n2, :],
                            w2[tap * 32:(tap + 1) * 32, :],
                            preferred_element_type=f32)
                acc2 = d if acc2 is None else acc2 + d
        y2 = jnp.maximum(acc2 + b2_ref[...], 0.0)
        y2 = jnp.concatenate([y2, jnp.zeros((CPAD, 64), f32)], axis=0)

        n3 = 7 * nbr
        w3 = w3_ref[...]
        acc3 = None
        for kh in range(3):
            for kw in range(3):
                start = kw * nbr + kh
                tap = kh * 3 + kw
                d = jnp.dot(y2[start:start + n3, :],
                            w3[tap * 64:(tap + 1) * 64, :],
                            preferred_element_type=f32)
                acc3 = d if acc3 is None else acc3 + d
        y3 = jnp.maximum(acc3 + b3_ref[...], 0.0)

        pieces = [y3[t * nbr:(t + 1) * nbr].reshape(nb, SB, 64)
                  for t in range(7)]
        feat = jnp.concatenate(
            [pieces[t][:, s:s + 1, :] for s in range(7) for t in range(7)],
            axis=1).reshape(nb, 49 * 64)
        feat = jnp.concatenate([feat, jnp.zeros((nb, 64), f32)], axis=1)

        h = jnp.maximum(
            jnp.dot(feat.astype(jnp.bfloat16), wh_ref[...],
                    preferred_element_type=f32) + bh_ref[...], 0.0)
        q = jnp.dot(h, wq_ref[...], preferred_element_type=f32) + bq_ref[...]
        o_ref[0] = q


def kernel(x_nchw, conv1_w, conv1_b, conv2_w, conv2_b, conv3_w, conv3_b,
           sel, wh, bh, wq, bq):
    B = x_nchw.shape[0]
    C = x_nchw.shape[1]
    A = wq.shape[1]
    nb = B // 2

    H, W = x_nchw.shape[2], x_nchw.shape[3]
    x = pl.pallas_call(
        functools.partial(_layout_kernel, nb=nb, C=C),
        out_shape=jax.ShapeDtypeStruct((2, nb, 12, 8, 3, 128), jnp.float32),
        grid=(2,),
        in_specs=[pl.BlockSpec((1, nb, C, H, W), lambda i: (i, 0, 0, 0, 0))],
        out_specs=pl.BlockSpec((1, nb, 12, 8, 3, 128),
                               lambda i: (i, 0, 0, 0, 0, 0)),
        compiler_params=pltpu.CompilerParams(
            dimension_semantics=("parallel",)),
    )(x_nchw.reshape(2, nb, C, H, W).astype(jnp.float32))
    x = x.reshape(2, nb * SB, 8, 3, 128)

    psel = np.zeros((48 * 32, 8 * 8 * C), np.float32)
    for r in range(8):
        for j, (_, _, dj, kh, pw) in enumerate(_TABLE[r]):
            t = r * 6 + j
            for w8 in range(8):
                kw = w8 + 8 * dj - 4 * pw
                if 0 <= kw < 8:
                    for c in range(C):
                        psel[t * 32 + c * 8 + w8, (kh * 8 + kw) * C + c] = 1.0

    args = (x, jnp.asarray(psel), conv1_w, conv1_b, conv2_w, conv2_b,
            conv3_w, conv3_b, wh, bh, wq, bq)
    in_specs = [
        pl.BlockSpec((1, nb * SB, 8, 3, 128), lambda i: (i, 0, 0, 0, 0)),
    ] + [pl.BlockSpec(a.shape, lambda i: (0,) * a.ndim) for a in args[1:]]

    n1 = 11 * nb * SB
    out = pl.pallas_call(
        functools.partial(_fused_kernel, nb=nb),
        out_shape=jax.ShapeDtypeStruct((2, nb, A), jnp.float32),
        grid=(2,),
        in_specs=in_specs,
        out_specs=pl.BlockSpec((1, nb, A), lambda i: (i, 0, 0)),
        compiler_params=pltpu.CompilerParams(
            dimension_semantics=("parallel",)),
    )(*args)
    return out.reshape(B, A)


# speedup vs baseline: 1.4929x; 1.3178x over previous
import functools

import numpy as np

import jax
import jax.numpy as jnp
from jax.experimental import pallas as pl
from jax.experimental.pallas import tpu as pltpu

SB = 12
CPAD = 16

def _class_taps():
    table = []
    for r in range(8):
        hp, dh = r // 4, r % 4
        entries = []
        for ph in range(2):
            kappa = (hp + ph) % 2
            delta = (ph + kappa) // 2
            kh = 4 * kappa + dh
            for pw, dj in ((0, 0), (1, 0), (1, 1)):
                entries.append((ph * 2 + pw, delta, dj, kh, pw))
        table.append(entries)
    return table

_TABLE = _class_taps()


def _layout_kernel(x_ref, m_ref, o_ref, *, nb, C):
    f32 = jnp.float32
    xw = None
    for c in range(C):
        d = jax.lax.dot_general(
            x_ref[0, :, c, :, :], m_ref[c],
            (((2,), (0,)), ((), ())),
            preferred_element_type=f32)
        xw = d if xw is None else xw + d
    xw = jnp.concatenate([xw, jnp.zeros((nb, 12, 384), f32)], axis=1)
    o_ref[0] = xw.reshape(nb, 12, 8, 3, 128)


def _fused_kernel(x_ref, p_ref, w1c_ref, b1_ref, w2_ref, b2_ref, w3_ref,
                  b3_ref, wh_ref, bh_ref, wq_ref, bq_ref, o_ref, *, nb):
    nbr = nb * SB
    n1 = 11 * nbr
    f32 = jnp.float32

    wtaps = jnp.dot(p_ref[...], w1c_ref[...],
                    preferred_element_type=f32)
    accs = [None, None, None, None]
    for kk in range(8):
        parts = [x_ref[0, :, kk, wp // 4, 32 * (wp % 4):32 * (wp % 4 + 1)]
                 for wp in range(12)]
        parts.append(jnp.zeros((CPAD, 32), f32))
        xflat = jnp.concatenate(parts, axis=0)
        for j, (ocls, delta, dj, _, _) in enumerate(_TABLE[kk]):
            shift = dj * nbr + delta
            t = kk * 6 + j
            d = jnp.dot(xflat[shift:shift + n1, :],
                        wtaps[t * 32:(t + 1) * 32, :],
                        preferred_element_type=f32)
            accs[ocls] = d if accs[ocls] is None else accs[ocls] + d

    if True:
        b1 = b1_ref[...]
        zpad1 = jnp.zeros((CPAD, 32), f32)
        y1_parts = []
        for a in accs:
            y1_parts.append(jnp.maximum(a + b1, 0.0))
            y1_parts.append(zpad1)
        y1 = jnp.concatenate(y1_parts, axis=0)
        cstride = n1 + CPAD

        n2 = 9 * nbr
        w2 = w2_ref[...]
        acc2 = None
        for kh in range(4):
            for kw in range(4):
                ph, a_ = kh % 2, kh // 2
                pw, b_ = kw % 2, kw // 2
                start = (ph * 2 + pw) * cstride + b_ * nbr + a_
                tap = kh * 4 + kw
                d = jnp.dot(y1[start:start + n2, :],
                            w2[tap * 32:(tap + 1) * 32, :],
                            preferred_element_type=f32)
                acc2 = d if acc2 is None else acc2 + d
        y2 = jnp.maximum(acc2 + b2_ref[...], 0.0)
        y2 = jnp.concatenate([y2, jnp.zeros((CPAD, 64), f32)], axis=0)

        n3 = 7 * nbr
        w3 = w3_ref[...]
        acc3 = None
        for kh in range(3):
            for kw in range(3):
                start = kw * nbr + kh
                tap = kh * 3 + kw
                d = jnp.dot(y2[start:start + n3, :],
                            w3[tap * 64:(tap + 1) * 64, :],
                            preferred_element_type=f32)
                acc3 = d if acc3 is None else acc3 + d
        y3 = jnp.maximum(acc3 + b3_ref[...], 0.0)

        pieces = [y3[t * nbr:(t + 1) * nbr].reshape(nb, SB, 64)
                  for t in range(7)]
        feat = jnp.concatenate(
            [pieces[t][:, s:s + 1, :] for s in range(7) for t in range(7)],
            axis=1).reshape(nb, 49 * 64)
        feat = jnp.concatenate([feat, jnp.zeros((nb, 64), f32)], axis=1)

        h = jnp.maximum(
            jnp.dot(feat.astype(jnp.bfloat16), wh_ref[...],
                    preferred_element_type=f32) + bh_ref[...], 0.0)
        q = jnp.dot(h, wq_ref[...], preferred_element_type=f32) + bq_ref[...]
        o_ref[0] = q


def kernel(x_nchw, conv1_w, conv1_b, conv2_w, conv2_b, conv3_w, conv3_b,
           sel, wh, bh, wq, bq):
    B = x_nchw.shape[0]
    C = x_nchw.shape[1]
    A = wq.shape[1]
    nb = B // 2

    H, W = x_nchw.shape[2], x_nchw.shape[3]
    msel = np.zeros((C, W, 384), np.float32)
    for c in range(C):
        for wp in range(12):
            for w8 in range(8):
                w = 8 * wp + w8
                if w < W:
                    msel[c, w, wp * 32 + c * 8 + w8] = 1.0
    x = pl.pallas_call(
        functools.partial(_layout_kernel, nb=nb, C=C),
        out_shape=jax.ShapeDtypeStruct((2, nb, 12, 8, 3, 128), jnp.float32),
        grid=(2,),
        in_specs=[pl.BlockSpec((1, nb, C, H, W), lambda i: (i, 0, 0, 0, 0)),
                  pl.BlockSpec((C, W, 384), lambda i: (0, 0, 0))],
        out_specs=pl.BlockSpec((1, nb, 12, 8, 3, 128),
                               lambda i: (i, 0, 0, 0, 0, 0)),
        compiler_params=pltpu.CompilerParams(
            dimension_semantics=("parallel",)),
    )(x_nchw.reshape(2, nb, C, H, W).astype(jnp.float32), jnp.asarray(msel))
    x = x.reshape(2, nb * SB, 8, 3, 128)

    psel = np.zeros((48 * 32, 8 * 8 * C), np.float32)
    for r in range(8):
        for j, (_, _, dj, kh, pw) in enumerate(_TABLE[r]):
            t = r * 6 + j
            for w8 in range(8):
                kw = w8 + 8 * dj - 4 * pw
                if 0 <= kw < 8:
                    for c in range(C):
                        psel[t * 32 + c * 8 + w8, (kh * 8 + kw) * C + c] = 1.0

    args = (x, jnp.asarray(psel), conv1_w, conv1_b, conv2_w, conv2_b,
            conv3_w, conv3_b, wh, bh, wq, bq)
    in_specs = [
        pl.BlockSpec((1, nb * SB, 8, 3, 128), lambda i: (i, 0, 0, 0, 0)),
    ] + [pl.BlockSpec(a.shape, lambda i: (0,) * a.ndim) for a in args[1:]]

    n1 = 11 * nb * SB
    out = pl.pallas_call(
        functools.partial(_fused_kernel, nb=nb),
        out_shape=jax.ShapeDtypeStruct((2, nb, A), jnp.float32),
        grid=(2,),
        in_specs=in_specs,
        out_specs=pl.BlockSpec((1, nb, A), lambda i: (i, 0, 0)),
        compiler_params=pltpu.CompilerParams(
            dimension_semantics=("parallel",)),
    )(*args)
    return out.reshape(B, A)
